# Initial kernel scaffold; baseline (speedup 1.0000x reference)
import jax
import jax.numpy as jnp
from jax.experimental import pallas as pl


def kernel(user_emb, item_emb, edge_index, edge_weight, noise):
    raise NotImplementedError("write your pallas kernel here")


# speedup vs baseline: 2.6219x; 2.6219x over previous
import functools

import jax
import jax.numpy as jnp
from jax import lax
from jax.experimental import pallas as pl
from jax.experimental.pallas import tpu as pltpu
from jax.experimental.pallas import tpu_sc as plsc

N_USERS = 50000
N_ITEMS = 50000
N = N_USERS + N_ITEMS
D = 64
E = 3200000
N_LAYERS = 3
EPS = 0.2
CL_LAYER = 0

NCHUNK = 4
LANES = 16
NTILES = 16
EDGES_PER_TILE = E // NTILES
CHUNK = 128
NFULL = EDGES_PER_TILE // CHUNK
REM = EDGES_PER_TILE - NFULL * CHUNK
N_PAD = 100096
ROWS_PER_TILE = N_PAD // NTILES
OUT_BLK = 368
NOUT = ROWS_PER_TILE // OUT_BLK


def _nn_body(noise_ref, out_ref):
    x = noise_ref[0]
    n = jnp.sqrt(jnp.sum(x * x, axis=-1, keepdims=True))
    y = x * (EPS / jnp.maximum(n, 1e-12))
    for c in range(NCHUNK):
        out_ref[0, c] = y[:, c * LANES:(c + 1) * LANES]


def _make_nn(rows_blk=2000):
    grid = (N_LAYERS, N // rows_blk)
    return pl.pallas_call(
        _nn_body,
        grid=grid,
        in_specs=[pl.BlockSpec((1, rows_blk, D), lambda k, r: (k, r, 0))],
        out_specs=pl.BlockSpec((1, NCHUNK, rows_blk, LANES),
                               lambda k, r: (k, 0, r, 0)),
        out_shape=jax.ShapeDtypeStruct((N_LAYERS, NCHUNK, N_PAD, LANES),
                                       jnp.float32),
    )


def _split_body(x_ref, out_ref):
    x = x_ref[...]
    for c in range(NCHUNK):
        out_ref[c] = x[:, c * LANES:(c + 1) * LANES]


def _make_split(rows_blk=2000):
    return pl.pallas_call(
        _split_body,
        grid=(N // rows_blk,),
        in_specs=[pl.BlockSpec((rows_blk, D), lambda r: (r, 0))],
        out_specs=pl.BlockSpec((NCHUNK, rows_blk, LANES), lambda r: (0, r, 0)),
        out_shape=jax.ShapeDtypeStruct((NCHUNK, N_PAD, LANES), jnp.float32),
    )


def _layer_body(emb_hbm, dst_hbm, src_hbm, w_hbm, nn_hbm, out_hbm,
                accum, srcv, dstv, wv, rowbuf, srcv2, dstv2, wv2, rowbuf2,
                zbuf, abuf, nbuf, sem):
    c = lax.axis_index("c")
    s = lax.axis_index("s")
    ebase = s * EDGES_PER_TILE
    rbase = s * ROWS_PER_TILE

    def zero_zbuf(i, _):
        zbuf[i] = jnp.zeros((LANES,), jnp.float32)
        return ()
    lax.fori_loop(0, OUT_BLK, zero_zbuf, ())

    def do_chunk(cc):
        def zero_step(i, _):
            pltpu.sync_copy(zbuf, accum.at[pl.ds(rbase + i * OUT_BLK, OUT_BLK)])
            return ()
        lax.fori_loop(0, NOUT, zero_step, ())
        plsc.subcore_barrier()

        def edge_step(i, _):
            off = ebase + i * CHUNK
            pltpu.sync_copy(dst_hbm.at[pl.ds(off, CHUNK)], dstv)
            pltpu.sync_copy(src_hbm.at[pl.ds(off, CHUNK)], srcv)
            pltpu.sync_copy(w_hbm.at[pl.ds(off, CHUNK)], wv)
            pltpu.async_copy(emb_hbm.at[cc].at[srcv], rowbuf, sem).wait()

            def scale(j, _):
                wvec = wv[pl.ds(j * LANES, LANES)]
                for l in range(LANES):
                    e = j * LANES + l
                    rowbuf[e] = rowbuf[e] * wvec[l]
                return ()
            lax.fori_loop(0, CHUNK // LANES, scale, ())
            pltpu.sync_copy(rowbuf, accum.at[dstv], add=True)
            return ()
        lax.fori_loop(0, NFULL, edge_step, ())

        off = ebase + NFULL * CHUNK
        pltpu.sync_copy(dst_hbm.at[pl.ds(off, REM)], dstv2)
        pltpu.sync_copy(src_hbm.at[pl.ds(off, REM)], srcv2)
        pltpu.sync_copy(w_hbm.at[pl.ds(off, REM)], wv2)
        pltpu.async_copy(emb_hbm.at[cc].at[srcv2], rowbuf2, sem).wait()

        def scale2(j, _):
            wvec = wv2[pl.ds(j * LANES, LANES)]
            for l in range(LANES):
                e = j * LANES + l
                rowbuf2[e] = rowbuf2[e] * wvec[l]
            return ()
        lax.fori_loop(0, REM // LANES, scale2, ())
        pltpu.sync_copy(rowbuf2, accum.at[dstv2], add=True)
        plsc.subcore_barrier()

        def out_step(i, _):
            r0 = rbase + i * OUT_BLK
            pltpu.sync_copy(accum.at[pl.ds(r0, OUT_BLK)], abuf)
            pltpu.sync_copy(nn_hbm.at[cc].at[pl.ds(r0, OUT_BLK)], nbuf)

            def perturb(e, _):
                a = abuf[e]
                abuf[e] = a + jnp.sign(a) * nbuf[e]
                return ()
            lax.fori_loop(0, OUT_BLK, perturb, ())
            pltpu.sync_copy(abuf, out_hbm.at[cc].at[pl.ds(r0, OUT_BLK)])
            return ()
        lax.fori_loop(0, NOUT, out_step, ())
        plsc.subcore_barrier()

    @pl.when(c == 0)
    def _():
        do_chunk(0)
        do_chunk(1)

    @pl.when(c == 1)
    def _():
        do_chunk(2)
        do_chunk(3)


def _make_layer():
    mesh = plsc.VectorSubcoreMesh(core_axis_name="c", subcore_axis_name="s")
    return pl.kernel(
        _layer_body,
        out_type=jax.ShapeDtypeStruct((NCHUNK, N_PAD, LANES), jnp.float32),
        mesh=mesh,
        compiler_params=pltpu.CompilerParams(use_tc_tiling_on_sc=False),
        scratch_types=[
            pltpu.MemorySpace.VMEM_SHARED((N_PAD, LANES), jnp.float32),
            pltpu.VMEM((CHUNK,), jnp.int32),
            pltpu.VMEM((CHUNK,), jnp.int32),
            pltpu.VMEM((CHUNK,), jnp.float32),
            pltpu.VMEM((CHUNK, LANES), jnp.float32),
            pltpu.VMEM((REM,), jnp.int32),
            pltpu.VMEM((REM,), jnp.int32),
            pltpu.VMEM((REM,), jnp.float32),
            pltpu.VMEM((REM, LANES), jnp.float32),
            pltpu.VMEM((OUT_BLK, LANES), jnp.float32),
            pltpu.VMEM((OUT_BLK, LANES), jnp.float32),
            pltpu.VMEM((OUT_BLK, LANES), jnp.float32),
            pltpu.SemaphoreType.DMA,
        ],
    )


def _final_body(e0_ref, e1_ref, e2_ref, e3_ref, out_ref):
    e0 = e0_ref[...]
    e1 = jnp.concatenate([e1_ref[c] for c in range(NCHUNK)], axis=-1)
    e2 = jnp.concatenate([e2_ref[c] for c in range(NCHUNK)], axis=-1)
    e3 = jnp.concatenate([e3_ref[c] for c in range(NCHUNK)], axis=-1)
    mean = (e0 + e1 + e2 + e3) * 0.25

    def l2(x):
        n = jnp.sqrt(jnp.sum(x * x, axis=-1, keepdims=True))
        return x / jnp.maximum(n, 1e-12)

    out_ref[0] = l2(mean)
    out_ref[1] = l2(e1)


def _make_final(rows_blk=2000):
    cspec = pl.BlockSpec((NCHUNK, rows_blk, LANES), lambda r: (0, r, 0))
    return pl.pallas_call(
        _final_body,
        grid=(N // rows_blk,),
        in_specs=[pl.BlockSpec((rows_blk, D), lambda r: (r, 0)),
                  cspec, cspec, cspec],
        out_specs=pl.BlockSpec((2, rows_blk, D), lambda r: (0, r, 0)),
        out_shape=jax.ShapeDtypeStruct((2, N, D), jnp.float32),
    )


@jax.jit
def kernel(user_emb, item_emb, edge_index, edge_weight, noise):
    emb0 = jnp.concatenate([user_emb, item_emb], axis=0)
    dst = edge_index[0]
    src = edge_index[1]

    nn = _make_nn()(noise)
    t0 = _make_split()(emb0)

    layer = _make_layer()
    tabs = [t0]
    for k in range(N_LAYERS):
        tabs.append(layer(tabs[-1], dst, src, edge_weight, nn[k]))

    return _make_final()(emb0, tabs[1], tabs[2], tabs[3])


# speedup vs baseline: 8.0448x; 3.0683x over previous
import functools

import jax
import jax.numpy as jnp
from jax import lax
from jax.experimental import pallas as pl
from jax.experimental.pallas import tpu as pltpu
from jax.experimental.pallas import tpu_sc as plsc

N_USERS = 50000
N_ITEMS = 50000
N = N_USERS + N_ITEMS
D = 64
E = 3200000
N_LAYERS = 3
EPS = 0.2
CL_LAYER = 0

NCHUNK = 4
LANES = 16
NTILES = 16
CHUNK = 128
NROWS = E // CHUNK
ROWS_T = NROWS // NTILES
PAIRS = 195
N_PAD = 100096
ROWS_PER_TILE = N_PAD // NTILES
OUT_BLK = 184
NOUT = ROWS_PER_TILE // OUT_BLK


def _nn_body(noise_ref, out_ref):
    x = noise_ref[0]
    n = jnp.sqrt(jnp.sum(x * x, axis=-1, keepdims=True))
    y = x * (EPS / jnp.maximum(n, 1e-12))
    for c in range(NCHUNK):
        out_ref[0, c] = y[:, c * LANES:(c + 1) * LANES]


def _make_nn(rows_blk=2000):
    grid = (N_LAYERS, N // rows_blk)
    return pl.pallas_call(
        _nn_body,
        grid=grid,
        in_specs=[pl.BlockSpec((1, rows_blk, D), lambda k, r: (k, r, 0))],
        out_specs=pl.BlockSpec((1, NCHUNK, rows_blk, LANES),
                               lambda k, r: (k, 0, r, 0)),
        out_shape=jax.ShapeDtypeStruct((N_LAYERS, NCHUNK, N_PAD, LANES),
                                       jnp.float32),
    )


def _split_body(x_ref, out_ref):
    x = x_ref[...]
    for c in range(NCHUNK):
        out_ref[c] = x[:, c * LANES:(c + 1) * LANES]


def _make_split(rows_blk=2000):
    return pl.pallas_call(
        _split_body,
        grid=(N // rows_blk,),
        in_specs=[pl.BlockSpec((rows_blk, D), lambda r: (r, 0))],
        out_specs=pl.BlockSpec((NCHUNK, rows_blk, LANES), lambda r: (0, r, 0)),
        out_shape=jax.ShapeDtypeStruct((NCHUNK, N_PAD, LANES), jnp.float32),
    )


def _layer_body(emb_hbm, dst_hbm, src_hbm, w_hbm, nn_hbm, out_hbm,
                accum, ring, dstm, srcm, wm, zbuf, abuf, nbuf,
                sem_g0, sem_g1, sem_s0, sem_s1):
    c = lax.axis_index("c")
    s = lax.axis_index("s")
    cbase = s * ROWS_T + jnp.minimum(s, 8)
    ntail = 2 + jnp.where(s < 8, 1, 0)
    rbase = s * ROWS_PER_TILE
    sem_g = (sem_g0, sem_g1)
    sem_s = (sem_s0, sem_s1)

    def zero_zbuf(i, _):
        zbuf[i] = jnp.zeros((LANES,), jnp.float32)
        return ()
    lax.fori_loop(0, OUT_BLK, zero_zbuf, ())

    def do_chunk(cc):
        tbl = emb_hbm.at[cc]

        def zero_step(i, _):
            pltpu.sync_copy(zbuf, accum.at[pl.ds(rbase + i * OUT_BLK, OUT_BLK)])
            return ()
        lax.fori_loop(0, NOUT, zero_step, ())
        plsc.subcore_barrier()

        def meta_load(q, h):
            r = cbase + q * 4
            pltpu.sync_copy(dst_hbm.at[pl.ds(r, 4)], dstm.at[h])
            pltpu.sync_copy(src_hbm.at[pl.ds(r, 4)], srcm.at[h])
            pltpu.sync_copy(w_hbm.at[pl.ds(r, 4)], wm.at[h])

        def fire_gathers(h):
            for j in range(4):
                pltpu.async_copy(tbl.at[srcm.at[h].at[j]],
                                 ring.at[h * 4 + j], sem_g[h])

        def wait_gathers(h):
            for j in range(4):
                pltpu.make_async_copy(tbl.at[pl.ds(0, CHUNK)],
                                      ring.at[h * 4 + j], sem_g[h]).wait()

        def scale(h):
            for j in range(4):
                slot = ring.at[h * 4 + j]

                def scale_group(g, _):
                    wvec = wm[h, j, pl.ds(g * LANES, LANES)]
                    for l in range(LANES):
                        e = g * LANES + l
                        slot[e] = slot[e] * wvec[l]
                    return ()
                lax.fori_loop(0, CHUNK // LANES, scale_group, ())

        def fire_scatters(h):
            for j in range(4):
                pltpu.async_copy(ring.at[h * 4 + j],
                                 accum.at[dstm.at[h].at[j]], sem_s[h],
                                 add=True)

        def drain_scatters(h):
            for j in range(4):
                pltpu.make_async_copy(ring.at[h * 4 + j],
                                      accum.at[pl.ds(0, CHUNK)],
                                      sem_s[h]).wait()

        meta_load(0, 0)
        fire_gathers(0)

        def pair_step(m, _):
            @pl.when(m > 0)
            def _():
                drain_scatters(1)
            meta_load(2 * m + 1, 1)
            fire_gathers(1)
            wait_gathers(0)
            scale(0)
            fire_scatters(0)
            drain_scatters(0)

            @pl.when(m < PAIRS - 1)
            def _():
                meta_load(2 * m + 2, 0)
                fire_gathers(0)
            wait_gathers(1)
            scale(1)
            fire_scatters(1)
            return ()
        lax.fori_loop(0, PAIRS, pair_step, ())
        drain_scatters(1)

        def tail_step(i, _):
            r = cbase + PAIRS * 8 + i
            pltpu.sync_copy(dst_hbm.at[pl.ds(r, 1)], dstm.at[0].at[pl.ds(0, 1)])
            pltpu.sync_copy(src_hbm.at[pl.ds(r, 1)], srcm.at[0].at[pl.ds(0, 1)])
            pltpu.sync_copy(w_hbm.at[pl.ds(r, 1)], wm.at[0].at[pl.ds(0, 1)])
            pltpu.async_copy(tbl.at[srcm.at[0].at[0]], ring.at[0],
                             sem_g[0]).wait()

            def scale_group(g, _):
                wvec = wm[0, 0, pl.ds(g * LANES, LANES)]
                for l in range(LANES):
                    e = g * LANES + l
                    ring[0, e] = ring[0, e] * wvec[l]
                return ()
            lax.fori_loop(0, CHUNK // LANES, scale_group, ())
            pltpu.sync_copy(ring.at[0], accum.at[dstm.at[0].at[0]], add=True)
            return ()
        lax.fori_loop(0, ntail, tail_step, ())
        plsc.subcore_barrier()

        def out_step(i, _):
            r0 = rbase + i * OUT_BLK
            pltpu.sync_copy(accum.at[pl.ds(r0, OUT_BLK)], abuf)
            pltpu.sync_copy(nn_hbm.at[cc].at[pl.ds(r0, OUT_BLK)], nbuf)

            def perturb(e, _):
                a = abuf[e]
                abuf[e] = a + jnp.sign(a) * nbuf[e]
                return ()
            lax.fori_loop(0, OUT_BLK, perturb, ())
            pltpu.sync_copy(abuf, out_hbm.at[cc].at[pl.ds(r0, OUT_BLK)])
            return ()
        lax.fori_loop(0, NOUT, out_step, ())
        plsc.subcore_barrier()

    @pl.when(c == 0)
    def _():
        do_chunk(0)
        do_chunk(1)

    @pl.when(c == 1)
    def _():
        do_chunk(2)
        do_chunk(3)


def _make_layer():
    mesh = plsc.VectorSubcoreMesh(core_axis_name="c", subcore_axis_name="s")
    return pl.kernel(
        _layer_body,
        out_type=jax.ShapeDtypeStruct((NCHUNK, N_PAD, LANES), jnp.float32),
        mesh=mesh,
        compiler_params=pltpu.CompilerParams(use_tc_tiling_on_sc=False),
        scratch_types=[
            pltpu.MemorySpace.VMEM_SHARED((N_PAD, LANES), jnp.float32),
            pltpu.VMEM((8, CHUNK, LANES), jnp.float32),
            pltpu.VMEM((2, 4, CHUNK), jnp.int32),
            pltpu.VMEM((2, 4, CHUNK), jnp.int32),
            pltpu.VMEM((2, 4, CHUNK), jnp.float32),
            pltpu.VMEM((OUT_BLK, LANES), jnp.float32),
            pltpu.VMEM((OUT_BLK, LANES), jnp.float32),
            pltpu.VMEM((OUT_BLK, LANES), jnp.float32),
            pltpu.SemaphoreType.DMA,
            pltpu.SemaphoreType.DMA,
            pltpu.SemaphoreType.DMA,
            pltpu.SemaphoreType.DMA,
        ],
    )


def _final_body(e0_ref, e1_ref, e2_ref, e3_ref, out_ref):
    e0 = e0_ref[...]
    e1 = jnp.concatenate([e1_ref[c] for c in range(NCHUNK)], axis=-1)
    e2 = jnp.concatenate([e2_ref[c] for c in range(NCHUNK)], axis=-1)
    e3 = jnp.concatenate([e3_ref[c] for c in range(NCHUNK)], axis=-1)
    mean = (e0 + e1 + e2 + e3) * 0.25

    def l2(x):
        n = jnp.sqrt(jnp.sum(x * x, axis=-1, keepdims=True))
        return x / jnp.maximum(n, 1e-12)

    out_ref[0] = l2(mean)
    out_ref[1] = l2(e1)


def _make_final(rows_blk=2000):
    cspec = pl.BlockSpec((NCHUNK, rows_blk, LANES), lambda r: (0, r, 0))
    return pl.pallas_call(
        _final_body,
        grid=(N // rows_blk,),
        in_specs=[pl.BlockSpec((rows_blk, D), lambda r: (r, 0)),
                  cspec, cspec, cspec],
        out_specs=pl.BlockSpec((2, rows_blk, D), lambda r: (0, r, 0)),
        out_shape=jax.ShapeDtypeStruct((2, N, D), jnp.float32),
    )


@jax.jit
def kernel(user_emb, item_emb, edge_index, edge_weight, noise):
    emb0 = jnp.concatenate([user_emb, item_emb], axis=0)
    dst = edge_index[0]
    src = edge_index[1]

    dst2d = dst.reshape(NROWS, CHUNK)
    src2d = src.reshape(NROWS, CHUNK)
    w2d = edge_weight.reshape(NROWS, CHUNK)

    nn = _make_nn()(noise)
    t0 = _make_split()(emb0)

    layer = _make_layer()
    tabs = [t0]
    for k in range(N_LAYERS):
        tabs.append(layer(tabs[-1], dst2d, src2d, w2d, nn[k]))

    return _make_final()(emb0, tabs[1], tabs[2], tabs[3])


# speedup vs baseline: 12.0527x; 1.4982x over previous
import functools

import jax
import jax.numpy as jnp
from jax import lax
from jax.experimental import pallas as pl
from jax.experimental.pallas import tpu as pltpu
from jax.experimental.pallas import tpu_sc as plsc

N_USERS = 50000
N_ITEMS = 50000
N = N_USERS + N_ITEMS
D = 64
E = 3200000
N_LAYERS = 3
EPS = 0.2
CL_LAYER = 0

NCHUNK = 4
LANES = 16
NTILES = 16
CHUNK = 128
NROWS = E // CHUNK
ROWS_T = NROWS // NTILES
NQ = 390
N_PAD = 100096
ROWS_PER_TILE = N_PAD // NTILES
OUT_BLK = 136
NOUT = ROWS_PER_TILE // OUT_BLK


def _nn_body(noise_ref, out_ref):
    x = noise_ref[0]
    n = jnp.sqrt(jnp.sum(x * x, axis=-1, keepdims=True))
    y = x * (EPS / jnp.maximum(n, 1e-12))
    for c in range(NCHUNK):
        out_ref[0, c] = y[:, c * LANES:(c + 1) * LANES]


def _make_nn(rows_blk=2000):
    grid = (N_LAYERS, N // rows_blk)
    return pl.pallas_call(
        _nn_body,
        grid=grid,
        in_specs=[pl.BlockSpec((1, rows_blk, D), lambda k, r: (k, r, 0))],
        out_specs=pl.BlockSpec((1, NCHUNK, rows_blk, LANES),
                               lambda k, r: (k, 0, r, 0)),
        out_shape=jax.ShapeDtypeStruct((N_LAYERS, NCHUNK, N_PAD, LANES),
                                       jnp.float32),
    )


def _split_body(x_ref, out_ref):
    x = x_ref[...]
    for c in range(NCHUNK):
        out_ref[c] = x[:, c * LANES:(c + 1) * LANES]


def _make_split(rows_blk=2000):
    return pl.pallas_call(
        _split_body,
        grid=(N // rows_blk,),
        in_specs=[pl.BlockSpec((rows_blk, D), lambda r: (r, 0))],
        out_specs=pl.BlockSpec((NCHUNK, rows_blk, LANES), lambda r: (0, r, 0)),
        out_shape=jax.ShapeDtypeStruct((NCHUNK, N_PAD, LANES), jnp.float32),
    )


def _layer_body(emb_hbm, meta_hbm, w_hbm, nn_hbm, out_hbm,
                accum, ring, meta, wm, zbuf, abuf, nbuf, sem_g, sem_s, sem_m):
    c = lax.axis_index("c")
    s = lax.axis_index("s")
    cbase = s * ROWS_T + jnp.minimum(s, 8)
    ntail = 2 + jnp.where(s < 8, 1, 0)
    rbase = s * ROWS_PER_TILE

    def zero_zbuf(i, _):
        zbuf[i] = jnp.zeros((LANES,), jnp.float32)
        return ()
    lax.fori_loop(0, OUT_BLK, zero_zbuf, ())

    def do_chunk(cc):
        tbl = emb_hbm.at[cc]

        def zero_step(i, _):
            pltpu.sync_copy(zbuf, accum.at[pl.ds(rbase + i * OUT_BLK, OUT_BLK)])
            return ()
        lax.fori_loop(0, NOUT, zero_step, ())
        plsc.subcore_barrier()

        def meta_issue(q):
            ms = lax.rem(q, 4)
            pltpu.async_copy(meta_hbm.at[pl.ds(cbase + q * 4, 4)],
                             meta.at[ms], sem_m)
            pltpu.async_copy(w_hbm.at[pl.ds(cbase + q * 4, 4)],
                             wm.at[ms], sem_m)

        def meta_wait():
            pltpu.make_async_copy(meta_hbm.at[pl.ds(0, 4)], meta.at[0],
                                  sem_m).wait()
            pltpu.make_async_copy(w_hbm.at[pl.ds(0, 4)], wm.at[0],
                                  sem_m).wait()

        def fire_gathers(q):
            ms = lax.rem(q, 4)
            hb = lax.rem(q, 2) * 4
            for j in range(4):
                pltpu.async_copy(tbl.at[meta.at[ms].at[j].at[1]],
                                 ring.at[hb + j], sem_g)

        def wait_gathers():
            for j in range(4):
                pltpu.make_async_copy(tbl.at[pl.ds(0, CHUNK)],
                                      ring.at[j], sem_g).wait()

        def scale(q):
            ms = lax.rem(q, 4)
            hb = lax.rem(q, 2) * 4
            for j in range(4):
                def scale_group(g, _):
                    wvec = wm[ms, j, pl.ds(g * LANES, LANES)]
                    for l in range(LANES):
                        e = g * LANES + l
                        ring[hb + j, e] = ring[hb + j, e] * wvec[l]
                    return ()
                lax.fori_loop(0, CHUNK // LANES, scale_group, ())

        def fire_scatters(q):
            ms = lax.rem(q, 4)
            hb = lax.rem(q, 2) * 4
            for j in range(4):
                pltpu.async_copy(ring.at[hb + j],
                                 accum.at[meta.at[ms].at[j].at[0]], sem_s,
                                 add=True)

        def drain_scatters():
            for j in range(4):
                pltpu.make_async_copy(ring.at[j], accum.at[pl.ds(0, CHUNK)],
                                      sem_s).wait()

        meta_issue(0)
        meta_wait()
        meta_issue(1)
        fire_gathers(0)

        def quad_step(q, _):
            @pl.when(q > 0)
            def _():
                drain_scatters()

            @pl.when(q + 1 < NQ)
            def _():
                meta_wait()

            @pl.when(q + 2 < NQ)
            def _():
                meta_issue(q + 2)
            wait_gathers()

            @pl.when(q + 1 < NQ)
            def _():
                fire_gathers(q + 1)
            scale(q)
            fire_scatters(q)
            return ()
        lax.fori_loop(0, NQ, quad_step, ())
        drain_scatters()

        def tail_step(i, _):
            r = cbase + NQ * 4 + i
            pltpu.sync_copy(meta_hbm.at[pl.ds(r, 1)], meta.at[0].at[pl.ds(0, 1)])
            pltpu.sync_copy(w_hbm.at[pl.ds(r, 1)], wm.at[0].at[pl.ds(0, 1)])
            pltpu.async_copy(tbl.at[meta.at[0].at[0].at[1]], ring.at[0],
                             sem_g).wait()

            def scale_group(g, _):
                wvec = wm[0, 0, pl.ds(g * LANES, LANES)]
                for l in range(LANES):
                    e = g * LANES + l
                    ring[0, e] = ring[0, e] * wvec[l]
                return ()
            lax.fori_loop(0, CHUNK // LANES, scale_group, ())
            pltpu.sync_copy(ring.at[0], accum.at[meta.at[0].at[0].at[0]],
                            add=True)
            return ()
        lax.fori_loop(0, ntail, tail_step, ())
        plsc.subcore_barrier()

        def out_step(i, _):
            r0 = rbase + i * OUT_BLK
            pltpu.sync_copy(accum.at[pl.ds(r0, OUT_BLK)], abuf)
            pltpu.sync_copy(nn_hbm.at[cc].at[pl.ds(r0, OUT_BLK)], nbuf)

            def perturb(e, _):
                a = abuf[e]
                abuf[e] = a + jnp.sign(a) * nbuf[e]
                return ()
            lax.fori_loop(0, OUT_BLK, perturb, ())
            pltpu.sync_copy(abuf, out_hbm.at[cc].at[pl.ds(r0, OUT_BLK)])
            return ()
        lax.fori_loop(0, NOUT, out_step, ())
        plsc.subcore_barrier()

    @pl.when(c == 0)
    def _():
        do_chunk(0)
        do_chunk(1)

    @pl.when(c == 1)
    def _():
        do_chunk(2)
        do_chunk(3)


def _make_layer():
    mesh = plsc.VectorSubcoreMesh(core_axis_name="c", subcore_axis_name="s")
    return pl.kernel(
        _layer_body,
        out_type=jax.ShapeDtypeStruct((NCHUNK, N_PAD, LANES), jnp.float32),
        mesh=mesh,
        compiler_params=pltpu.CompilerParams(use_tc_tiling_on_sc=False),
        scratch_types=[
            pltpu.MemorySpace.VMEM_SHARED((N_PAD, LANES), jnp.float32),
            pltpu.VMEM((8, CHUNK, LANES), jnp.float32),
            pltpu.VMEM((4, 4, 2, CHUNK), jnp.int32),
            pltpu.VMEM((4, 4, CHUNK), jnp.float32),
            pltpu.VMEM((OUT_BLK, LANES), jnp.float32),
            pltpu.VMEM((OUT_BLK, LANES), jnp.float32),
            pltpu.VMEM((OUT_BLK, LANES), jnp.float32),
            pltpu.SemaphoreType.DMA,
            pltpu.SemaphoreType.DMA,
            pltpu.SemaphoreType.DMA,
        ],
    )


def _final_body(e0_ref, e1_ref, e2_ref, e3_ref, out_ref):
    e0 = e0_ref[...]
    e1 = jnp.concatenate([e1_ref[c] for c in range(NCHUNK)], axis=-1)
    e2 = jnp.concatenate([e2_ref[c] for c in range(NCHUNK)], axis=-1)
    e3 = jnp.concatenate([e3_ref[c] for c in range(NCHUNK)], axis=-1)
    mean = (e0 + e1 + e2 + e3) * 0.25

    def l2(x):
        n = jnp.sqrt(jnp.sum(x * x, axis=-1, keepdims=True))
        return x / jnp.maximum(n, 1e-12)

    out_ref[0] = l2(mean)
    out_ref[1] = l2(e1)


def _make_final(rows_blk=2000):
    cspec = pl.BlockSpec((NCHUNK, rows_blk, LANES), lambda r: (0, r, 0))
    return pl.pallas_call(
        _final_body,
        grid=(N // rows_blk,),
        in_specs=[pl.BlockSpec((rows_blk, D), lambda r: (r, 0)),
                  cspec, cspec, cspec],
        out_specs=pl.BlockSpec((2, rows_blk, D), lambda r: (0, r, 0)),
        out_shape=jax.ShapeDtypeStruct((2, N, D), jnp.float32),
    )


@jax.jit
def kernel(user_emb, item_emb, edge_index, edge_weight, noise):
    emb0 = jnp.concatenate([user_emb, item_emb], axis=0)
    dst = edge_index[0]
    src = edge_index[1]

    meta = jnp.stack([dst.reshape(NROWS, CHUNK),
                      src.reshape(NROWS, CHUNK)],
                     axis=1)
    w2d = edge_weight.reshape(NROWS, CHUNK)

    nn = _make_nn()(noise)
    t0 = _make_split()(emb0)

    layer = _make_layer()
    tabs = [t0]
    for k in range(N_LAYERS):
        tabs.append(layer(tabs[-1], meta, w2d, nn[k]))

    return _make_final()(emb0, tabs[1], tabs[2], tabs[3])


# speedup vs baseline: 12.6035x; 1.0457x over previous
import functools

import jax
import jax.numpy as jnp
from jax import lax
from jax.experimental import pallas as pl
from jax.experimental.pallas import tpu as pltpu
from jax.experimental.pallas import tpu_sc as plsc

N_USERS = 50000
N_ITEMS = 50000
N = N_USERS + N_ITEMS
D = 64
E = 3200000
N_LAYERS = 3
EPS = 0.2
CL_LAYER = 0

NCHUNK = 4
LANES = 16
NTILES = 16
CHUNK = 128
NROWS = E // CHUNK
ROWS_T = NROWS // NTILES
NQ = 390
N_PAD = 100096
ROWS_PER_TILE = N_PAD // NTILES
OUT_BLK = 136
NOUT = ROWS_PER_TILE // OUT_BLK


def _nn_body(noise_ref, out_ref):
    x = noise_ref[0]
    n = jnp.sqrt(jnp.sum(x * x, axis=-1, keepdims=True))
    y = x * (EPS / jnp.maximum(n, 1e-12))
    for c in range(NCHUNK):
        out_ref[0, c] = y[:, c * LANES:(c + 1) * LANES]


def _make_nn(rows_blk=2000):
    grid = (N_LAYERS, N // rows_blk)
    return pl.pallas_call(
        _nn_body,
        grid=grid,
        in_specs=[pl.BlockSpec((1, rows_blk, D), lambda k, r: (k, r, 0))],
        out_specs=pl.BlockSpec((1, NCHUNK, rows_blk, LANES),
                               lambda k, r: (k, 0, r, 0)),
        out_shape=jax.ShapeDtypeStruct((N_LAYERS, NCHUNK, N_PAD, LANES),
                                       jnp.float32),
    )


def _split_body(x_ref, out_ref):
    x = x_ref[...]
    for c in range(NCHUNK):
        out_ref[c] = x[:, c * LANES:(c + 1) * LANES]


def _make_split(rows_blk=2000):
    return pl.pallas_call(
        _split_body,
        grid=(N // rows_blk,),
        in_specs=[pl.BlockSpec((rows_blk, D), lambda r: (r, 0))],
        out_specs=pl.BlockSpec((NCHUNK, rows_blk, LANES), lambda r: (0, r, 0)),
        out_shape=jax.ShapeDtypeStruct((NCHUNK, N_PAD, LANES), jnp.float32),
    )


def _layer_body(emb_hbm, meta_hbm, w_hbm, nn_hbm, out_hbm,
                accum, ring, meta, wm, zbuf, abuf, nbuf, sem_g, sem_s, sem_m):
    c = lax.axis_index("c")
    s = lax.axis_index("s")
    cbase = s * ROWS_T + jnp.minimum(s, 8)
    ntail = 2 + jnp.where(s < 8, 1, 0)
    rbase = s * ROWS_PER_TILE

    def zero_zbuf(i, _):
        zbuf[i] = jnp.zeros((LANES,), jnp.float32)
        return ()
    lax.fori_loop(0, OUT_BLK, zero_zbuf, ())

    def do_chunk(cc):
        tbl = emb_hbm.at[cc]

        def zero_step(i, _):
            pltpu.sync_copy(zbuf, accum.at[pl.ds(rbase + i * OUT_BLK, OUT_BLK)])
            return ()
        lax.fori_loop(0, NOUT, zero_step, ())
        plsc.subcore_barrier()

        def meta_issue(q):
            ms = lax.rem(q, 4)
            pltpu.async_copy(meta_hbm.at[pl.ds(cbase + q * 4, 4)],
                             meta.at[ms], sem_m)
            pltpu.async_copy(w_hbm.at[pl.ds(cbase + q * 4, 4)],
                             wm.at[ms], sem_m)

        def meta_wait():
            pltpu.make_async_copy(meta_hbm.at[pl.ds(0, 4)], meta.at[0],
                                  sem_m).wait()
            pltpu.make_async_copy(w_hbm.at[pl.ds(0, 4)], wm.at[0],
                                  sem_m).wait()

        def fire_gathers(q):
            ms = lax.rem(q, 4)
            hb = lax.rem(q, 2) * 4
            for j in range(4):
                pltpu.async_copy(tbl.at[meta.at[ms].at[j].at[1]],
                                 ring.at[hb + j], sem_g)

        def wait_gathers():
            for j in range(4):
                pltpu.make_async_copy(tbl.at[pl.ds(0, CHUNK)],
                                      ring.at[j], sem_g).wait()

        def scale(q):
            ms = lax.rem(q, 4)
            hb = lax.rem(q, 2) * 4
            for j in range(4):
                @plsc.parallel_loop(0, CHUNK // LANES, unroll=2)
                def scale_group(g):
                    wvec = wm[ms, j, pl.ds(g * LANES, LANES)]
                    for l in range(LANES):
                        e = g * LANES + l
                        ring[hb + j, e] = ring[hb + j, e] * wvec[l]

        def fire_scatters(q):
            ms = lax.rem(q, 4)
            hb = lax.rem(q, 2) * 4
            for j in range(4):
                pltpu.async_copy(ring.at[hb + j],
                                 accum.at[meta.at[ms].at[j].at[0]], sem_s,
                                 add=True)

        def drain_scatters():
            for j in range(4):
                pltpu.make_async_copy(ring.at[j], accum.at[pl.ds(0, CHUNK)],
                                      sem_s).wait()

        meta_issue(0)
        meta_wait()
        meta_issue(1)
        fire_gathers(0)

        def quad_step(q, _):
            @pl.when(q > 0)
            def _():
                drain_scatters()

            @pl.when(q + 1 < NQ)
            def _():
                meta_wait()

            @pl.when(q + 2 < NQ)
            def _():
                meta_issue(q + 2)
            wait_gathers()

            @pl.when(q + 1 < NQ)
            def _():
                fire_gathers(q + 1)
            scale(q)
            fire_scatters(q)
            return ()
        lax.fori_loop(0, NQ, quad_step, ())
        drain_scatters()

        def tail_step(i, _):
            r = cbase + NQ * 4 + i
            pltpu.sync_copy(meta_hbm.at[pl.ds(r, 1)], meta.at[0].at[pl.ds(0, 1)])
            pltpu.sync_copy(w_hbm.at[pl.ds(r, 1)], wm.at[0].at[pl.ds(0, 1)])
            pltpu.async_copy(tbl.at[meta.at[0].at[0].at[1]], ring.at[0],
                             sem_g).wait()

            def scale_group(g, _):
                wvec = wm[0, 0, pl.ds(g * LANES, LANES)]
                for l in range(LANES):
                    e = g * LANES + l
                    ring[0, e] = ring[0, e] * wvec[l]
                return ()
            lax.fori_loop(0, CHUNK // LANES, scale_group, ())
            pltpu.sync_copy(ring.at[0], accum.at[meta.at[0].at[0].at[0]],
                            add=True)
            return ()
        lax.fori_loop(0, ntail, tail_step, ())
        plsc.subcore_barrier()

        def out_step(i, _):
            r0 = rbase + i * OUT_BLK
            pltpu.sync_copy(accum.at[pl.ds(r0, OUT_BLK)], abuf)
            pltpu.sync_copy(nn_hbm.at[cc].at[pl.ds(r0, OUT_BLK)], nbuf)

            @plsc.parallel_loop(0, OUT_BLK, unroll=4)
            def perturb(e):
                a = abuf[e]
                abuf[e] = a + jnp.sign(a) * nbuf[e]
            pltpu.sync_copy(abuf, out_hbm.at[cc].at[pl.ds(r0, OUT_BLK)])
            return ()
        lax.fori_loop(0, NOUT, out_step, ())
        plsc.subcore_barrier()

    @pl.when(c == 0)
    def _():
        do_chunk(0)
        do_chunk(1)

    @pl.when(c == 1)
    def _():
        do_chunk(2)
        do_chunk(3)


def _make_layer():
    mesh = plsc.VectorSubcoreMesh(core_axis_name="c", subcore_axis_name="s")
    return pl.kernel(
        _layer_body,
        out_type=jax.ShapeDtypeStruct((NCHUNK, N_PAD, LANES), jnp.float32),
        mesh=mesh,
        compiler_params=pltpu.CompilerParams(use_tc_tiling_on_sc=False),
        scratch_types=[
            pltpu.MemorySpace.VMEM_SHARED((N_PAD, LANES), jnp.float32),
            pltpu.VMEM((8, CHUNK, LANES), jnp.float32),
            pltpu.VMEM((4, 4, 2, CHUNK), jnp.int32),
            pltpu.VMEM((4, 4, CHUNK), jnp.float32),
            pltpu.VMEM((OUT_BLK, LANES), jnp.float32),
            pltpu.VMEM((OUT_BLK, LANES), jnp.float32),
            pltpu.VMEM((OUT_BLK, LANES), jnp.float32),
            pltpu.SemaphoreType.DMA,
            pltpu.SemaphoreType.DMA,
            pltpu.SemaphoreType.DMA,
        ],
    )


def _final_body(e0_ref, e1_ref, e2_ref, e3_ref, out_ref):
    e0 = e0_ref[...]
    e1 = jnp.concatenate([e1_ref[c] for c in range(NCHUNK)], axis=-1)
    e2 = jnp.concatenate([e2_ref[c] for c in range(NCHUNK)], axis=-1)
    e3 = jnp.concatenate([e3_ref[c] for c in range(NCHUNK)], axis=-1)
    mean = (e0 + e1 + e2 + e3) * 0.25

    def l2(x):
        n = jnp.sqrt(jnp.sum(x * x, axis=-1, keepdims=True))
        return x / jnp.maximum(n, 1e-12)

    out_ref[0] = l2(mean)
    out_ref[1] = l2(e1)


def _make_final(rows_blk=2000):
    cspec = pl.BlockSpec((NCHUNK, rows_blk, LANES), lambda r: (0, r, 0))
    return pl.pallas_call(
        _final_body,
        grid=(N // rows_blk,),
        in_specs=[pl.BlockSpec((rows_blk, D), lambda r: (r, 0)),
                  cspec, cspec, cspec],
        out_specs=pl.BlockSpec((2, rows_blk, D), lambda r: (0, r, 0)),
        out_shape=jax.ShapeDtypeStruct((2, N, D), jnp.float32),
    )


@jax.jit
def kernel(user_emb, item_emb, edge_index, edge_weight, noise):
    emb0 = jnp.concatenate([user_emb, item_emb], axis=0)
    dst = edge_index[0]
    src = edge_index[1]

    meta = jnp.stack([dst.reshape(NROWS, CHUNK),
                      src.reshape(NROWS, CHUNK)],
                     axis=1)
    w2d = edge_weight.reshape(NROWS, CHUNK)

    nn = _make_nn()(noise)
    t0 = _make_split()(emb0)

    layer = _make_layer()
    tabs = [t0]
    for k in range(N_LAYERS):
        tabs.append(layer(tabs[-1], meta, w2d, nn[k]))

    return _make_final()(emb0, tabs[1], tabs[2], tabs[3])


# speedup vs baseline: 12.7124x; 1.0086x over previous
import functools

import jax
import jax.numpy as jnp
from jax import lax
from jax.experimental import pallas as pl
from jax.experimental.pallas import tpu as pltpu
from jax.experimental.pallas import tpu_sc as plsc

N_USERS = 50000
N_ITEMS = 50000
N = N_USERS + N_ITEMS
D = 64
E = 3200000
N_LAYERS = 3
EPS = 0.2
CL_LAYER = 0

NCHUNK = 4
LANES = 16
NTILES = 16
CHUNK = 128
NROWS = E // CHUNK
ROWS_T = NROWS // NTILES
NQ = 390
N_PAD = 100096
ROWS_PER_TILE = N_PAD // NTILES
OUT_BLK = 184
NOUT = ROWS_PER_TILE // OUT_BLK


def _nn_body(noise_ref, out_ref):
    x = noise_ref[0]
    n = jnp.sqrt(jnp.sum(x * x, axis=-1, keepdims=True))
    y = x * (EPS / jnp.maximum(n, 1e-12))
    for c in range(NCHUNK):
        out_ref[0, c] = y[:, c * LANES:(c + 1) * LANES]


def _make_nn(rows_blk=2000):
    grid = (N_LAYERS, N // rows_blk)
    return pl.pallas_call(
        _nn_body,
        grid=grid,
        in_specs=[pl.BlockSpec((1, rows_blk, D), lambda k, r: (k, r, 0))],
        out_specs=pl.BlockSpec((1, NCHUNK, rows_blk, LANES),
                               lambda k, r: (k, 0, r, 0)),
        out_shape=jax.ShapeDtypeStruct((N_LAYERS, NCHUNK, N_PAD, LANES),
                                       jnp.float32),
    )


def _split_body(x_ref, out_ref):
    x = x_ref[...]
    for c in range(NCHUNK):
        out_ref[c] = x[:, c * LANES:(c + 1) * LANES]


def _make_split(rows_blk=2000):
    return pl.pallas_call(
        _split_body,
        grid=(N // rows_blk,),
        in_specs=[pl.BlockSpec((rows_blk, D), lambda r: (r, 0))],
        out_specs=pl.BlockSpec((NCHUNK, rows_blk, LANES), lambda r: (0, r, 0)),
        out_shape=jax.ShapeDtypeStruct((NCHUNK, N_PAD, LANES), jnp.float32),
    )


def _layer_body(emb_hbm, meta_hbm, w_hbm, nn_hbm, out_hbm,
                accum, ring, meta, wm, abuf, nbuf, sem_g, sem_s, sem_m):
    c = lax.axis_index("c")
    s = lax.axis_index("s")
    cbase = s * ROWS_T + jnp.minimum(s, 8)
    ntail = 2 + jnp.where(s < 8, 1, 0)
    rbase = s * ROWS_PER_TILE

    def do_chunk(cc):
        tbl = emb_hbm.at[cc]

        def zero_abuf(i, _):
            abuf[i] = jnp.zeros((LANES,), jnp.float32)
            return ()
        lax.fori_loop(0, OUT_BLK, zero_abuf, ())

        def zero_step(i, _):
            pltpu.sync_copy(abuf, accum.at[pl.ds(rbase + i * OUT_BLK, OUT_BLK)])
            return ()
        lax.fori_loop(0, NOUT, zero_step, ())
        plsc.subcore_barrier()

        def meta_issue(q):
            ms = lax.rem(q, 4)
            pltpu.async_copy(meta_hbm.at[pl.ds(cbase + q * 4, 4)],
                             meta.at[ms], sem_m)
            pltpu.async_copy(w_hbm.at[pl.ds(cbase + q * 4, 4)],
                             wm.at[ms], sem_m)

        def meta_wait():
            pltpu.make_async_copy(meta_hbm.at[pl.ds(0, 4)], meta.at[0],
                                  sem_m).wait()
            pltpu.make_async_copy(w_hbm.at[pl.ds(0, 4)], wm.at[0],
                                  sem_m).wait()

        def fire_gathers(q):
            ms = lax.rem(q, 4)
            hb = lax.rem(q, 2) * 4
            for j in range(4):
                pltpu.async_copy(tbl.at[meta.at[ms].at[j].at[1]],
                                 ring.at[hb + j], sem_g)

        def wait_gathers():
            for j in range(4):
                pltpu.make_async_copy(tbl.at[pl.ds(0, CHUNK)],
                                      ring.at[j], sem_g).wait()

        def scale(q):
            ms = lax.rem(q, 4)
            hb = lax.rem(q, 2) * 4
            for j in range(4):
                @plsc.parallel_loop(0, CHUNK // LANES, unroll=2)
                def scale_group(g):
                    wvec = wm[ms, j, pl.ds(g * LANES, LANES)]
                    for l in range(LANES):
                        e = g * LANES + l
                        ring[hb + j, e] = ring[hb + j, e] * wvec[l]

        def fire_scatters(q):
            ms = lax.rem(q, 4)
            hb = lax.rem(q, 2) * 4
            for j in range(4):
                pltpu.async_copy(ring.at[hb + j],
                                 accum.at[meta.at[ms].at[j].at[0]], sem_s,
                                 add=True)

        def drain_scatters():
            for j in range(4):
                pltpu.make_async_copy(ring.at[j], accum.at[pl.ds(0, CHUNK)],
                                      sem_s).wait()

        meta_issue(0)
        meta_wait()
        meta_issue(1)
        fire_gathers(0)

        def quad_step(q, _):
            @pl.when(q > 0)
            def _():
                drain_scatters()

            @pl.when(q + 1 < NQ)
            def _():
                meta_wait()

            @pl.when(q + 2 < NQ)
            def _():
                meta_issue(q + 2)
            wait_gathers()

            @pl.when(q + 1 < NQ)
            def _():
                fire_gathers(q + 1)
            scale(q)
            fire_scatters(q)
            return ()
        lax.fori_loop(0, NQ, quad_step, ())
        drain_scatters()

        def tail_step(i, _):
            r = cbase + NQ * 4 + i
            pltpu.sync_copy(meta_hbm.at[pl.ds(r, 1)], meta.at[0].at[pl.ds(0, 1)])
            pltpu.sync_copy(w_hbm.at[pl.ds(r, 1)], wm.at[0].at[pl.ds(0, 1)])
            pltpu.async_copy(tbl.at[meta.at[0].at[0].at[1]], ring.at[0],
                             sem_g).wait()

            def scale_group(g, _):
                wvec = wm[0, 0, pl.ds(g * LANES, LANES)]
                for l in range(LANES):
                    e = g * LANES + l
                    ring[0, e] = ring[0, e] * wvec[l]
                return ()
            lax.fori_loop(0, CHUNK // LANES, scale_group, ())
            pltpu.sync_copy(ring.at[0], accum.at[meta.at[0].at[0].at[0]],
                            add=True)
            return ()
        lax.fori_loop(0, ntail, tail_step, ())
        plsc.subcore_barrier()

        def out_step(i, _):
            r0 = rbase + i * OUT_BLK
            pltpu.sync_copy(accum.at[pl.ds(r0, OUT_BLK)], abuf)
            pltpu.sync_copy(nn_hbm.at[cc].at[pl.ds(r0, OUT_BLK)], nbuf)

            @plsc.parallel_loop(0, OUT_BLK, unroll=4)
            def perturb(e):
                a = abuf[e]
                abuf[e] = a + jnp.sign(a) * nbuf[e]
            pltpu.sync_copy(abuf, out_hbm.at[cc].at[pl.ds(r0, OUT_BLK)])
            return ()
        lax.fori_loop(0, NOUT, out_step, ())
        plsc.subcore_barrier()

    @pl.when(c == 0)
    def _():
        do_chunk(0)
        do_chunk(1)

    @pl.when(c == 1)
    def _():
        do_chunk(2)
        do_chunk(3)


def _make_layer():
    mesh = plsc.VectorSubcoreMesh(core_axis_name="c", subcore_axis_name="s")
    return pl.kernel(
        _layer_body,
        out_type=jax.ShapeDtypeStruct((NCHUNK, N_PAD, LANES), jnp.float32),
        mesh=mesh,
        compiler_params=pltpu.CompilerParams(use_tc_tiling_on_sc=False),
        scratch_types=[
            pltpu.MemorySpace.VMEM_SHARED((N_PAD, LANES), jnp.float32),
            pltpu.VMEM((8, CHUNK, LANES), jnp.float32),
            pltpu.VMEM((4, 4, 2, CHUNK), jnp.int32),
            pltpu.VMEM((4, 4, CHUNK), jnp.float32),
            pltpu.VMEM((OUT_BLK, LANES), jnp.float32),
            pltpu.VMEM((OUT_BLK, LANES), jnp.float32),
            pltpu.SemaphoreType.DMA,
            pltpu.SemaphoreType.DMA,
            pltpu.SemaphoreType.DMA,
        ],
    )


def _final_body(e0_ref, e1_ref, e2_ref, e3_ref, out_ref):
    e0 = e0_ref[...]
    e1 = jnp.concatenate([e1_ref[c] for c in range(NCHUNK)], axis=-1)
    e2 = jnp.concatenate([e2_ref[c] for c in range(NCHUNK)], axis=-1)
    e3 = jnp.concatenate([e3_ref[c] for c in range(NCHUNK)], axis=-1)
    mean = (e0 + e1 + e2 + e3) * 0.25

    def l2(x):
        n = jnp.sqrt(jnp.sum(x * x, axis=-1, keepdims=True))
        return x / jnp.maximum(n, 1e-12)

    out_ref[0] = l2(mean)
    out_ref[1] = l2(e1)


def _make_final(rows_blk=2000):
    cspec = pl.BlockSpec((NCHUNK, rows_blk, LANES), lambda r: (0, r, 0))
    return pl.pallas_call(
        _final_body,
        grid=(N // rows_blk,),
        in_specs=[pl.BlockSpec((rows_blk, D), lambda r: (r, 0)),
                  cspec, cspec, cspec],
        out_specs=pl.BlockSpec((2, rows_blk, D), lambda r: (0, r, 0)),
        out_shape=jax.ShapeDtypeStruct((2, N, D), jnp.float32),
    )


@jax.jit
def kernel(user_emb, item_emb, edge_index, edge_weight, noise):
    emb0 = jnp.concatenate([user_emb, item_emb], axis=0)
    dst = edge_index[0]
    src = edge_index[1]

    meta = jnp.stack([dst.reshape(NROWS, CHUNK),
                      src.reshape(NROWS, CHUNK)],
                     axis=1)
    w2d = edge_weight.reshape(NROWS, CHUNK)

    nn = _make_nn()(noise)
    t0 = _make_split()(emb0)

    layer = _make_layer()
    tabs = [t0]
    for k in range(N_LAYERS):
        tabs.append(layer(tabs[-1], meta, w2d, nn[k]))

    return _make_final()(emb0, tabs[1], tabs[2], tabs[3])


# speedup vs baseline: 12.7402x; 1.0022x over previous
import functools

import jax
import jax.numpy as jnp
from jax import lax
from jax.experimental import pallas as pl
from jax.experimental.pallas import tpu as pltpu
from jax.experimental.pallas import tpu_sc as plsc

N_USERS = 50000
N_ITEMS = 50000
N = N_USERS + N_ITEMS
D = 64
E = 3200000
N_LAYERS = 3
EPS = 0.2
CL_LAYER = 0

NCHUNK = 4
LANES = 16
NTILES = 16
CHUNK = 128
NROWS = E // CHUNK
ROWS_T = NROWS // NTILES
NQ = 390
N_PAD = 100096
ROWS_PER_TILE = N_PAD // NTILES
OUT_BLK = 184
NOUT = ROWS_PER_TILE // OUT_BLK


def _nn_body(noise_ref, out_ref):
    x = noise_ref[0]
    n = jnp.sqrt(jnp.sum(x * x, axis=-1, keepdims=True))
    y = x * (EPS / jnp.maximum(n, 1e-12))
    for c in range(NCHUNK):
        out_ref[0, c] = y[:, c * LANES:(c + 1) * LANES]


def _make_nn(rows_blk=2000):
    grid = (N_LAYERS, N // rows_blk)
    return pl.pallas_call(
        _nn_body,
        grid=grid,
        in_specs=[pl.BlockSpec((1, rows_blk, D), lambda k, r: (k, r, 0))],
        out_specs=pl.BlockSpec((1, NCHUNK, rows_blk, LANES),
                               lambda k, r: (k, 0, r, 0)),
        out_shape=jax.ShapeDtypeStruct((N_LAYERS, NCHUNK, N_PAD, LANES),
                                       jnp.float32),
    )


def _split_body(x_ref, out_ref):
    x = x_ref[...]
    for c in range(NCHUNK):
        out_ref[c] = x[:, c * LANES:(c + 1) * LANES]


def _make_split(rows_blk=2000):
    return pl.pallas_call(
        _split_body,
        grid=(N // rows_blk,),
        in_specs=[pl.BlockSpec((rows_blk, D), lambda r: (r, 0))],
        out_specs=pl.BlockSpec((NCHUNK, rows_blk, LANES), lambda r: (0, r, 0)),
        out_shape=jax.ShapeDtypeStruct((NCHUNK, N_PAD, LANES), jnp.float32),
    )


def _layer_body(emb_hbm, meta_hbm, w_hbm, nn_hbm, out_hbm,
                accum, ring, meta, wm, abuf, nbuf, sem_g, sem_s, sem_m):
    c = lax.axis_index("c")
    s = lax.axis_index("s")
    cbase = s * ROWS_T + jnp.minimum(s, 8)
    ntail = 2 + jnp.where(s < 8, 1, 0)
    rbase = s * ROWS_PER_TILE

    def do_chunk(cc):
        tbl = emb_hbm.at[cc]

        def zero_abuf(i, _):
            abuf[i] = jnp.zeros((LANES,), jnp.float32)
            return ()
        lax.fori_loop(0, OUT_BLK, zero_abuf, ())

        def zero_step(i, _):
            pltpu.sync_copy(abuf, accum.at[pl.ds(rbase + i * OUT_BLK, OUT_BLK)])
            return ()
        lax.fori_loop(0, NOUT, zero_step, ())
        plsc.subcore_barrier()

        def meta_issue(q):
            ms = lax.rem(q, 4)
            pltpu.async_copy(meta_hbm.at[pl.ds(cbase + q * 4, 4)],
                             meta.at[ms], sem_m)
            pltpu.async_copy(w_hbm.at[pl.ds(cbase + q * 4, 4)],
                             wm.at[ms], sem_m)

        def meta_wait():
            pltpu.make_async_copy(meta_hbm.at[pl.ds(0, 4)], meta.at[0],
                                  sem_m).wait()
            pltpu.make_async_copy(w_hbm.at[pl.ds(0, 4)], wm.at[0],
                                  sem_m).wait()

        def fire_gathers(q):
            ms = lax.rem(q, 4)
            hb = lax.rem(q, 2) * 4
            for j in range(4):
                pltpu.async_copy(tbl.at[meta.at[ms].at[j].at[1]],
                                 ring.at[hb + j], sem_g)

        def wait_gathers():
            for j in range(4):
                pltpu.make_async_copy(tbl.at[pl.ds(0, CHUNK)],
                                      ring.at[j], sem_g).wait()

        def scale(q):
            ms = lax.rem(q, 4)
            hb = lax.rem(q, 2) * 4
            for j in range(4):
                @plsc.parallel_loop(0, CHUNK // LANES, unroll=4)
                def scale_group(g):
                    wvec = wm[ms, j, pl.ds(g * LANES, LANES)]
                    for l in range(LANES):
                        e = g * LANES + l
                        ring[hb + j, e] = ring[hb + j, e] * wvec[l]

        def fire_scatters(q):
            ms = lax.rem(q, 4)
            hb = lax.rem(q, 2) * 4
            for j in range(4):
                pltpu.async_copy(ring.at[hb + j],
                                 accum.at[meta.at[ms].at[j].at[0]], sem_s,
                                 add=True)

        def drain_scatters():
            for j in range(4):
                pltpu.make_async_copy(ring.at[j], accum.at[pl.ds(0, CHUNK)],
                                      sem_s).wait()

        meta_issue(0)
        meta_wait()
        meta_issue(1)
        fire_gathers(0)

        def quad_step(q, _):
            @pl.when(q > 0)
            def _():
                drain_scatters()

            @pl.when(q + 1 < NQ)
            def _():
                meta_wait()

            @pl.when(q + 2 < NQ)
            def _():
                meta_issue(q + 2)
            wait_gathers()

            @pl.when(q + 1 < NQ)
            def _():
                fire_gathers(q + 1)
            scale(q)
            fire_scatters(q)
            return ()
        lax.fori_loop(0, NQ, quad_step, ())
        drain_scatters()

        def tail_step(i, _):
            r = cbase + NQ * 4 + i
            pltpu.sync_copy(meta_hbm.at[pl.ds(r, 1)], meta.at[0].at[pl.ds(0, 1)])
            pltpu.sync_copy(w_hbm.at[pl.ds(r, 1)], wm.at[0].at[pl.ds(0, 1)])
            pltpu.async_copy(tbl.at[meta.at[0].at[0].at[1]], ring.at[0],
                             sem_g).wait()

            def scale_group(g, _):
                wvec = wm[0, 0, pl.ds(g * LANES, LANES)]
                for l in range(LANES):
                    e = g * LANES + l
                    ring[0, e] = ring[0, e] * wvec[l]
                return ()
            lax.fori_loop(0, CHUNK // LANES, scale_group, ())
            pltpu.sync_copy(ring.at[0], accum.at[meta.at[0].at[0].at[0]],
                            add=True)
            return ()
        lax.fori_loop(0, ntail, tail_step, ())
        plsc.subcore_barrier()

        def out_step(i, _):
            r0 = rbase + i * OUT_BLK
            pltpu.sync_copy(accum.at[pl.ds(r0, OUT_BLK)], abuf)
            pltpu.sync_copy(nn_hbm.at[cc].at[pl.ds(r0, OUT_BLK)], nbuf)

            @plsc.parallel_loop(0, OUT_BLK, unroll=4)
            def perturb(e):
                a = abuf[e]
                abuf[e] = a + jnp.sign(a) * nbuf[e]
            pltpu.sync_copy(abuf, out_hbm.at[cc].at[pl.ds(r0, OUT_BLK)])
            return ()
        lax.fori_loop(0, NOUT, out_step, ())
        plsc.subcore_barrier()

    @pl.when(c == 0)
    def _():
        do_chunk(0)
        do_chunk(1)

    @pl.when(c == 1)
    def _():
        do_chunk(2)
        do_chunk(3)


def _make_layer():
    mesh = plsc.VectorSubcoreMesh(core_axis_name="c", subcore_axis_name="s")
    return pl.kernel(
        _layer_body,
        out_type=jax.ShapeDtypeStruct((NCHUNK, N_PAD, LANES), jnp.float32),
        mesh=mesh,
        compiler_params=pltpu.CompilerParams(use_tc_tiling_on_sc=False),
        scratch_types=[
            pltpu.MemorySpace.VMEM_SHARED((N_PAD, LANES), jnp.float32),
            pltpu.VMEM((8, CHUNK, LANES), jnp.float32),
            pltpu.VMEM((4, 4, 2, CHUNK), jnp.int32),
            pltpu.VMEM((4, 4, CHUNK), jnp.float32),
            pltpu.VMEM((OUT_BLK, LANES), jnp.float32),
            pltpu.VMEM((OUT_BLK, LANES), jnp.float32),
            pltpu.SemaphoreType.DMA,
            pltpu.SemaphoreType.DMA,
            pltpu.SemaphoreType.DMA,
        ],
    )


def _final_body(e0_ref, e1_ref, e2_ref, e3_ref, out_ref):
    e0 = e0_ref[...]
    e1 = jnp.concatenate([e1_ref[c] for c in range(NCHUNK)], axis=-1)
    e2 = jnp.concatenate([e2_ref[c] for c in range(NCHUNK)], axis=-1)
    e3 = jnp.concatenate([e3_ref[c] for c in range(NCHUNK)], axis=-1)
    mean = (e0 + e1 + e2 + e3) * 0.25

    def l2(x):
        n = jnp.sqrt(jnp.sum(x * x, axis=-1, keepdims=True))
        return x / jnp.maximum(n, 1e-12)

    out_ref[0] = l2(mean)
    out_ref[1] = l2(e1)


def _make_final(rows_blk=2000):
    cspec = pl.BlockSpec((NCHUNK, rows_blk, LANES), lambda r: (0, r, 0))
    return pl.pallas_call(
        _final_body,
        grid=(N // rows_blk,),
        in_specs=[pl.BlockSpec((rows_blk, D), lambda r: (r, 0)),
                  cspec, cspec, cspec],
        out_specs=pl.BlockSpec((2, rows_blk, D), lambda r: (0, r, 0)),
        out_shape=jax.ShapeDtypeStruct((2, N, D), jnp.float32),
    )


@jax.jit
def kernel(user_emb, item_emb, edge_index, edge_weight, noise):
    emb0 = jnp.concatenate([user_emb, item_emb], axis=0)
    dst = edge_index[0]
    src = edge_index[1]

    meta = jnp.stack([dst.reshape(NROWS, CHUNK),
                      src.reshape(NROWS, CHUNK)],
                     axis=1)
    w2d = edge_weight.reshape(NROWS, CHUNK)

    nn = _make_nn()(noise)
    t0 = _make_split()(emb0)

    layer = _make_layer()
    tabs = [t0]
    for k in range(N_LAYERS):
        tabs.append(layer(tabs[-1], meta, w2d, nn[k]))

    return _make_final()(emb0, tabs[1], tabs[2], tabs[3])
